# baseline (speedup 1.0000x reference)
import functools

import jax
import jax.numpy as jnp
from jax import lax
from jax.experimental import pallas as pl
from jax.experimental.pallas import tpu as pltpu

_EPS = 1e-5


def _fold(sum_row, sq_row, gamma, beta, inv_count):
    mean = sum_row * inv_count
    var = sq_row * inv_count - mean * mean
    scale = gamma * lax.rsqrt(var + _EPS)
    shift = beta - mean * scale
    return scale, shift


def _shift_w(h, shift, outw):
    rows, win, c = h.shape
    left = max(0, -shift)
    start = max(0, shift)
    stop = min(win, outw + shift)
    take = h[:, start:stop, :]
    right = outw - left - (stop - start)
    parts = []
    if left:
        parts.append(jnp.zeros((rows, left, c), h.dtype))
    parts.append(take)
    if right:
        parts.append(jnp.zeros((rows, right, c), h.dtype))
    return jnp.concatenate(parts, axis=1) if len(parts) > 1 else take


def _stats_kernel(x_ref, s_ref, q_ref):
    c = x_ref.shape[-1]
    xv = x_ref[...].reshape(-1, c)
    s_ref[0] = jnp.sum(xv, axis=0, keepdims=True)
    q_ref[0] = jnp.sum(xv * xv, axis=0, keepdims=True)


def _conv1_kernel(x_ref, sum_ref, sq_ref, bnx_ref, w1pk_ref,
                  c1_ref, s1_ref, s2_ref, *, inv_count, H, W, Wp):
    cin = x_ref.shape[-1]
    planes = w1pk_ref.shape[-1]
    bnx = bnx_ref[...]
    scale, shift = _fold(sum_ref[...], sq_ref[...], bnx[0:1], bnx[1:2],
                         inv_count)
    h = jnp.maximum(
        x_ref[0] * scale.reshape(1, 1, cin) + shift.reshape(1, 1, cin), 0.0)
    hb = h.astype(jnp.bfloat16)
    hx = jnp.concatenate([_shift_w(hb, dx - 1, Wp) for dx in range(3)],
                         axis=2)
    z = jnp.zeros((1, Wp, 3 * cin), jnp.bfloat16)
    hx = jnp.concatenate([z, hx, z], axis=0)
    acc = None
    for dy in range(3):
        lhs = hx[dy:dy + H].reshape(H * Wp, 3 * cin)
        d = jnp.dot(lhs, w1pk_ref[dy], preferred_element_type=jnp.float32)
        acc = d if acc is None else acc + d
    acc = acc.reshape(H, Wp, planes)
    col = lax.broadcasted_iota(jnp.int32, acc.shape, 1)
    acc = jnp.where(col < W, acc, 0.0)
    c1_ref[...] = acc[None].astype(jnp.bfloat16)
    a2 = acc.reshape(H * Wp, planes)
    s1_ref[0] = jnp.sum(a2, axis=0, keepdims=True)
    s2_ref[0] = jnp.sum(a2 * a2, axis=0, keepdims=True)


def _psa_kernel(c1_ref, x_ref, s1_ref, s2_ref, sx_ref, qx_ref, bn2_ref,
                bnx_ref, wpk_ref, ball_ref, wsc_ref, w1e_ref, b1e_ref,
                w2e_ref, b2e_ref, smat_ref, o_ref, *, inv1, inv2, Ho, Wo, Wp):
    C = ball_ref.shape[-1]
    cin = x_ref.shape[-1]
    bn2 = bn2_ref[...]
    scale2, shift2 = _fold(s1_ref[...], s2_ref[...], bn2[0:1], bn2[1:2], inv2)
    c1 = c1_ref[0]
    h2 = jnp.maximum(
        c1 * scale2.reshape(1, 1, C) + shift2.reshape(1, 1, C), 0.0)
    col = lax.broadcasted_iota(jnp.int32, h2.shape, 1)
    h2 = jnp.where(col < Wo, h2, 0.0).astype(jnp.bfloat16)
    hx = jnp.concatenate([_shift_w(h2, dx - 4, Wp) for dx in range(9)],
                         axis=2)
    z = jnp.zeros((4, Wp, 9 * C), jnp.bfloat16)
    hx = jnp.concatenate([z, hx, z], axis=0)

    ds = []
    for dy in range(9):
        nc = 32 * (min(dy, 8 - dy, 3) + 1)
        lhs = hx[dy:dy + Ho].reshape(Ho * Wp, 9 * C)
        ds.append(jnp.dot(lhs, wpk_ref[dy][:, :nc],
                          preferred_element_type=jnp.float32))
    f128 = ds[3] + ds[4] + ds[5]
    f96 = ds[2] + ds[6]
    f64 = ds[1] + ds[7]
    f32b = ds[0] + ds[8]
    feats = jnp.concatenate(
        [f128[:, :32] + f96[:, :32] + f64[:, :32] + f32b,
         f128[:, 32:64] + f96[:, 32:64] + f64[:, 32:64],
         f128[:, 64:96] + f96[:, 64:96],
         f128[:, 96:128]], axis=1) + ball_ref[...]

    fm = feats.reshape(Ho, Wp, C)
    colf = lax.broadcasted_iota(jnp.int32, fm.shape, 1)
    pooled = (jnp.sum(jnp.where(colf < Wo, fm, 0.0), axis=(0, 1))
              .reshape(1, C) * (1.0 / (Ho * Wo)))
    zf = jnp.maximum(
        jnp.dot(pooled, w1e_ref[...], preferred_element_type=jnp.float32)
        + b1e_ref[...], 0.0)
    logits = (jnp.dot(zf, w2e_ref[...], preferred_element_type=jnp.float32)
              + b2e_ref[...])
    se = 1.0 / (1.0 + jnp.exp(-logits))
    e = jnp.exp(se)
    denom = jnp.dot(e, smat_ref[...], preferred_element_type=jnp.float32)
    att = e / denom

    scale_s, shift_s = _fold(sx_ref[...], qx_ref[...], bnx_ref[2:3],
                             bnx_ref[3:4], inv1)
    xw = jnp.concatenate(
        [x_ref[0], jnp.zeros((Ho, Wp - Wo, cin), jnp.float32)], axis=1)
    hs = jnp.maximum(
        xw * scale_s.reshape(1, 1, cin) + shift_s.reshape(1, 1, cin), 0.0)
    sc = jnp.dot(hs.astype(jnp.bfloat16).reshape(Ho * Wp, cin), wsc_ref[...],
                 preferred_element_type=jnp.float32)
    outw = (feats * att + sc).reshape(Ho, Wp, C)
    o_ref[...] = outw[:, :Wo, :][None]


def kernel(x, w1, w_sc, bn_x, bn2, wall, ball, w1e, b1e, w2e, b2e, smat):
    n, H, W, cin = x.shape
    planes = w1.shape[-1]
    Ho, Wo = H, W
    Wp = 64
    w1pk = w1.reshape(3, 3 * cin, planes).astype(jnp.bfloat16)
    wallpk = wall.reshape(9, 9 * planes, planes).astype(jnp.bfloat16)
    wscb = w_sc.astype(jnp.bfloat16)

    ps, pq = pl.pallas_call(
        _stats_kernel,
        out_shape=(jax.ShapeDtypeStruct((n, 1, cin), jnp.float32),
                   jax.ShapeDtypeStruct((n, 1, cin), jnp.float32)),
        grid=(n,),
        in_specs=[pl.BlockSpec((1, H, W, cin), lambda i: (i, 0, 0, 0))],
        out_specs=(pl.BlockSpec((1, 1, cin), lambda i: (i, 0, 0)),
                   pl.BlockSpec((1, 1, cin), lambda i: (i, 0, 0))),
        compiler_params=pltpu.CompilerParams(
            dimension_semantics=("parallel",)),
    )(x)
    sum_x = jnp.sum(ps, axis=0)
    sq_x = jnp.sum(pq, axis=0)

    conv1 = functools.partial(_conv1_kernel, inv_count=1.0 / (n * H * W),
                              H=H, W=W, Wp=Wp)
    c1, p1, p2 = pl.pallas_call(
        conv1,
        out_shape=(jax.ShapeDtypeStruct((n, H, Wp, planes), jnp.bfloat16),
                   jax.ShapeDtypeStruct((n, 1, planes), jnp.float32),
                   jax.ShapeDtypeStruct((n, 1, planes), jnp.float32)),
        grid=(n,),
        in_specs=[
            pl.BlockSpec((1, H, W, cin), lambda i: (i, 0, 0, 0)),
            pl.BlockSpec((1, cin), lambda i: (0, 0)),
            pl.BlockSpec((1, cin), lambda i: (0, 0)),
            pl.BlockSpec(bn_x.shape, lambda i: (0, 0)),
            pl.BlockSpec((3, 3 * cin, planes), lambda i: (0, 0, 0)),
        ],
        out_specs=(pl.BlockSpec((1, H, Wp, planes), lambda i: (i, 0, 0, 0)),
                   pl.BlockSpec((1, 1, planes), lambda i: (i, 0, 0)),
                   pl.BlockSpec((1, 1, planes), lambda i: (i, 0, 0))),
        compiler_params=pltpu.CompilerParams(
            dimension_semantics=("parallel",)),
    )(x, sum_x, sq_x, bn_x, w1pk)
    s1 = jnp.sum(p1, axis=0)
    s2 = jnp.sum(p2, axis=0)

    psa = functools.partial(_psa_kernel, inv1=1.0 / (n * H * W),
                            inv2=1.0 / (n * Ho * Wo), Ho=Ho, Wo=Wo, Wp=Wp)
    out = pl.pallas_call(
        psa,
        out_shape=jax.ShapeDtypeStruct((n, Ho, Wo, planes), jnp.float32),
        grid=(n,),
        in_specs=[
            pl.BlockSpec((1, Ho, Wp, planes), lambda i: (i, 0, 0, 0)),
            pl.BlockSpec((1, H, W, cin), lambda i: (i, 0, 0, 0)),
            pl.BlockSpec((1, planes), lambda i: (0, 0)),
            pl.BlockSpec((1, planes), lambda i: (0, 0)),
            pl.BlockSpec((1, cin), lambda i: (0, 0)),
            pl.BlockSpec((1, cin), lambda i: (0, 0)),
            pl.BlockSpec((2, planes), lambda i: (0, 0)),
            pl.BlockSpec(bn_x.shape, lambda i: (0, 0)),
            pl.BlockSpec((9, 9 * planes, planes), lambda i: (0, 0, 0)),
            pl.BlockSpec((1, planes), lambda i: (0, 0)),
            pl.BlockSpec((cin, planes), lambda i: (0, 0)),
            pl.BlockSpec(w1e.shape, lambda i: (0, 0)),
            pl.BlockSpec(b1e.shape, lambda i: (0, 0)),
            pl.BlockSpec(w2e.shape, lambda i: (0, 0)),
            pl.BlockSpec(b2e.shape, lambda i: (0, 0)),
            pl.BlockSpec(smat.shape, lambda i: (0, 0)),
        ],
        out_specs=pl.BlockSpec((1, Ho, Wo, planes), lambda i: (i, 0, 0, 0)),
        compiler_params=pltpu.CompilerParams(
            dimension_semantics=("parallel",)),
    )(c1, x, s1, s2, sum_x, sq_x, bn2, bn_x, wallpk, ball, wscb,
      w1e, b1e, w2e, b2e, smat)
    return out


# speedup vs baseline: 2.0097x; 2.0097x over previous
import functools

import jax
import jax.numpy as jnp
from jax import lax
from jax.experimental import pallas as pl
from jax.experimental.pallas import tpu as pltpu

_EPS = 1e-5
_DY_ORDER = (3, 4, 5, 2, 0, 6, 1, 7, 8)


def _fold(sum_row, sq_row, gamma, beta, inv_count):
    mean = sum_row * inv_count
    var = sq_row * inv_count - mean * mean
    scale = gamma * lax.rsqrt(var + _EPS)
    shift = beta - mean * scale
    return scale, shift


def _shift_w(h, shift, outw):
    rows, win, c = h.shape
    left = max(0, -shift)
    start = max(0, shift)
    stop = min(win, outw + shift)
    take = h[:, start:stop, :]
    right = outw - left - (stop - start)
    parts = []
    if left:
        parts.append(jnp.zeros((rows, left, c), h.dtype))
    parts.append(take)
    if right:
        parts.append(jnp.zeros((rows, right, c), h.dtype))
    return jnp.concatenate(parts, axis=1) if len(parts) > 1 else take


def _stats_kernel(x_ref, s_ref, q_ref):
    c = x_ref.shape[-1]
    xv = x_ref[...].reshape(-1, c)
    s_ref[0] = jnp.sum(xv, axis=0, keepdims=True)
    q_ref[0] = jnp.sum(xv * xv, axis=0, keepdims=True)


def _conv1_compute(xv, sum_row, sq_row, bnx, w1pk, inv_count, H, W):
    cin = xv.shape[-1]
    planes = w1pk.shape[-1] // 3
    scale, shift = _fold(sum_row, sq_row, bnx[0:1], bnx[1:2], inv_count)
    h = jnp.maximum(
        xv * scale.reshape(1, 1, cin) + shift.reshape(1, 1, cin), 0.0)
    hb = h.astype(jnp.bfloat16)
    hx = jnp.concatenate([_shift_w(hb, dx - 1, W) for dx in range(3)],
                         axis=2)
    z = jnp.zeros((1, W, 3 * cin), jnp.bfloat16)
    hx = jnp.concatenate([z, hx, z], axis=0)
    G = jnp.dot(hx.reshape((H + 2) * W, 3 * cin), w1pk,
                preferred_element_type=jnp.float32)
    return (G[0:H * W, 0:planes] + G[W:W + H * W, planes:2 * planes]
            + G[2 * W:2 * W + H * W, 2 * planes:3 * planes])


def _conv1_kernel(x_ref, sum_ref, sq_ref, bnx_ref, w1pk_ref,
                  c1_ref, s1_ref, s2_ref, *, inv_count, H, W):
    planes = w1pk_ref.shape[-1] // 3
    s1 = s2 = None
    for b in range(x_ref.shape[0]):
        acc = _conv1_compute(x_ref[b], sum_ref[...], sq_ref[...],
                             bnx_ref[...], w1pk_ref[...], inv_count, H, W)
        c1_ref[b] = acc.reshape(H, W, planes).astype(jnp.bfloat16)
        ps = jnp.sum(acc, axis=0, keepdims=True)
        pq = jnp.sum(acc * acc, axis=0, keepdims=True)
        s1 = ps if s1 is None else s1 + ps
        s2 = pq if s2 is None else s2 + pq
    s1_ref[0] = s1
    s2_ref[0] = s2


def _psa_kernel(c1_ref, x_ref, s1_ref, s2_ref, sx_ref, qx_ref, bn2_ref,
                bnx_ref, wpk_ref, ball_ref, wsc_ref, w1e_ref,
                b1e_ref, w2e_ref, b2e_ref, smat_ref, o_ref,
                *, inv1, inv2, Ho, Wo):
    C = ball_ref.shape[-1]
    cin = x_ref.shape[-1]
    Wp = Wo
    c1 = c1_ref[0]
    bn2 = bn2_ref[...]
    scale2, shift2 = _fold(s1_ref[...], s2_ref[...], bn2[0:1], bn2[1:2], inv2)
    h2 = jnp.maximum(
        c1 * scale2.reshape(1, 1, C) + shift2.reshape(1, 1, C),
        0.0).astype(jnp.bfloat16)
    hx = jnp.concatenate([_shift_w(h2, dx - 4, Wp) for dx in range(9)],
                         axis=2)
    z = jnp.zeros((4, Wp, 9 * C), jnp.bfloat16)
    hx = jnp.concatenate([z, hx, z], axis=0)

    lhs = hx.reshape((Ho + 8) * Wp, 9 * C)
    ds = {}
    off = 0
    for g in range(3):
        G = jnp.dot(lhs, wpk_ref[:, 256 * g:256 * (g + 1)],
                    preferred_element_type=jnp.float32)
        goff = off
        while off < 256 * (g + 1):
            dy = _DY_ORDER[len(ds)]
            nc = 32 * (min(dy, 8 - dy, 3) + 1)
            ds[dy] = G[dy * Wp:dy * Wp + Ho * Wp, off - goff:off - goff + nc]
            off += nc
    f128 = ds[3] + ds[4] + ds[5]
    f96 = ds[2] + ds[6]
    f64 = ds[1] + ds[7]
    f32b = ds[0] + ds[8]
    feats = jnp.concatenate(
        [f128[:, :32] + f96[:, :32] + f64[:, :32] + f32b,
         f128[:, 32:64] + f96[:, 32:64] + f64[:, 32:64],
         f128[:, 64:96] + f96[:, 64:96],
         f128[:, 96:128]], axis=1) + ball_ref[...]

    pooled = (jnp.sum(feats, axis=0, keepdims=True) * (1.0 / (Ho * Wo)))
    zf = jnp.maximum(
        jnp.dot(pooled, w1e_ref[...], preferred_element_type=jnp.float32)
        + b1e_ref[...], 0.0)
    logits = (jnp.dot(zf, w2e_ref[...], preferred_element_type=jnp.float32)
              + b2e_ref[...])
    se = 1.0 / (1.0 + jnp.exp(-logits))
    e = jnp.exp(se)
    denom = jnp.dot(e, smat_ref[...], preferred_element_type=jnp.float32)
    att = e / denom

    scale_s, shift_s = _fold(sx_ref[...], qx_ref[...], bnx_ref[2:3],
                             bnx_ref[3:4], inv1)
    hs = jnp.maximum(
        x_ref[0] * scale_s.reshape(1, 1, cin) + shift_s.reshape(1, 1, cin),
        0.0)
    sc = jnp.dot(hs.astype(jnp.bfloat16).reshape(Ho * Wo, cin), wsc_ref[...],
                 preferred_element_type=jnp.float32)
    o_ref[...] = (feats * att + sc).reshape(1, Ho, Wo, C)


def kernel(x, w1, w_sc, bn_x, bn2, wall, ball, w1e, b1e, w2e, b2e, smat):
    n, H, W, cin = x.shape
    planes = w1.shape[-1]
    Ho, Wo = H, W
    w1r = w1.reshape(3, 3 * cin, planes)
    w1pk = jnp.concatenate([w1r[0], w1r[1], w1r[2]],
                           axis=1).astype(jnp.bfloat16)
    wall9 = wall.reshape(9, 9 * planes, planes)
    wallpk = jnp.concatenate(
        [wall9[dy][:, :32 * (min(dy, 8 - dy, 3) + 1)] for dy in _DY_ORDER],
        axis=1).astype(jnp.bfloat16)
    wscb = w_sc.astype(jnp.bfloat16)

    bs = 16 if n % 16 == 0 else 1
    ps, pq = pl.pallas_call(
        _stats_kernel,
        out_shape=(jax.ShapeDtypeStruct((n // bs, 1, cin), jnp.float32),
                   jax.ShapeDtypeStruct((n // bs, 1, cin), jnp.float32)),
        grid=(n // bs,),
        in_specs=[pl.BlockSpec((bs, H, W, cin), lambda i: (i, 0, 0, 0))],
        out_specs=(pl.BlockSpec((1, 1, cin), lambda i: (i, 0, 0)),
                   pl.BlockSpec((1, 1, cin), lambda i: (i, 0, 0))),
        compiler_params=pltpu.CompilerParams(
            dimension_semantics=("parallel",)),
    )(x)
    sum_x = jnp.sum(ps, axis=0)
    sq_x = jnp.sum(pq, axis=0)

    conv1 = functools.partial(_conv1_kernel, inv_count=1.0 / (n * H * W),
                              H=H, W=W)
    cb = 2 if n % 2 == 0 else 1
    c1, p1, p2 = pl.pallas_call(
        conv1,
        out_shape=(jax.ShapeDtypeStruct((n, H, W, planes), jnp.bfloat16),
                   jax.ShapeDtypeStruct((n // cb, 1, planes), jnp.float32),
                   jax.ShapeDtypeStruct((n // cb, 1, planes), jnp.float32)),
        grid=(n // cb,),
        in_specs=[
            pl.BlockSpec((cb, H, W, cin), lambda i: (i, 0, 0, 0)),
            pl.BlockSpec((1, cin), lambda i: (0, 0)),
            pl.BlockSpec((1, cin), lambda i: (0, 0)),
            pl.BlockSpec(bn_x.shape, lambda i: (0, 0)),
            pl.BlockSpec((3 * cin, 3 * planes), lambda i: (0, 0)),
        ],
        out_specs=(pl.BlockSpec((cb, H, W, planes), lambda i: (i, 0, 0, 0)),
                   pl.BlockSpec((1, 1, planes), lambda i: (i, 0, 0)),
                   pl.BlockSpec((1, 1, planes), lambda i: (i, 0, 0))),
        compiler_params=pltpu.CompilerParams(
            dimension_semantics=("parallel",)),
    )(x, sum_x, sq_x, bn_x, w1pk)
    s1 = jnp.sum(p1, axis=0)
    s2 = jnp.sum(p2, axis=0)

    psa = functools.partial(_psa_kernel, inv1=1.0 / (n * H * W),
                            inv2=1.0 / (n * Ho * Wo), Ho=Ho, Wo=Wo)
    out = pl.pallas_call(
        psa,
        out_shape=jax.ShapeDtypeStruct((n, Ho, Wo, planes), jnp.float32),
        grid=(n,),
        in_specs=[
            pl.BlockSpec((1, Ho, Wo, planes), lambda i: (i, 0, 0, 0)),
            pl.BlockSpec((1, H, W, cin), lambda i: (i, 0, 0, 0)),
            pl.BlockSpec((1, planes), lambda i: (0, 0)),
            pl.BlockSpec((1, planes), lambda i: (0, 0)),
            pl.BlockSpec((1, cin), lambda i: (0, 0)),
            pl.BlockSpec((1, cin), lambda i: (0, 0)),
            pl.BlockSpec((2, planes), lambda i: (0, 0)),
            pl.BlockSpec(bn_x.shape, lambda i: (0, 0)),
            pl.BlockSpec((9 * planes, 6 * planes), lambda i: (0, 0)),
            pl.BlockSpec((1, planes), lambda i: (0, 0)),
            pl.BlockSpec((cin, planes), lambda i: (0, 0)),
            pl.BlockSpec(w1e.shape, lambda i: (0, 0)),
            pl.BlockSpec(b1e.shape, lambda i: (0, 0)),
            pl.BlockSpec(w2e.shape, lambda i: (0, 0)),
            pl.BlockSpec(b2e.shape, lambda i: (0, 0)),
            pl.BlockSpec(smat.shape, lambda i: (0, 0)),
        ],
        out_specs=pl.BlockSpec((1, Ho, Wo, planes), lambda i: (i, 0, 0, 0)),
        compiler_params=pltpu.CompilerParams(
            dimension_semantics=("parallel",)),
    )(c1, x, s1, s2, sum_x, sq_x, bn2, bn_x, wallpk, ball, wscb,
      w1e, b1e, w2e, b2e, smat)
    return out
